# Initial kernel scaffold; baseline (speedup 1.0000x reference)
import jax
import jax.numpy as jnp
from jax.experimental import pallas as pl


def kernel(batch, embeddings):
    raise NotImplementedError("write your pallas kernel here")


# speedup vs baseline: 1.6470x; 1.6470x over previous
import jax
import jax.numpy as jnp
from jax.experimental import pallas as pl

NUM_EMB = 1024
EMB_DIM = 256
ROWS = 16 * 1024
TILE = 1024


def _vq_body(x_ref, e_ref, enc_ref, out_ref):
    x = x_ref[:]
    emb = e_ref[:]
    sim = jnp.dot(x, emb, preferred_element_type=jnp.float32)
    e2 = jnp.sum(emb * emb, axis=0)
    x2 = jnp.sum(x * x, axis=1, keepdims=True)
    dist = x2 + e2[None, :] - 2.0 * sim
    m = jnp.min(dist, axis=1, keepdims=True)
    iota = jax.lax.broadcasted_iota(jnp.int32, dist.shape, 1)
    idx = jnp.min(jnp.where(dist == m, iota, NUM_EMB), axis=1, keepdims=True)
    onehot = (iota == idx).astype(jnp.float32)
    enc_ref[:] = onehot
    out_ref[:] = jax.lax.dot_general(
        onehot, emb, (((1,), (1,)), ((), ())),
        preferred_element_type=jnp.float32)


def kernel(batch, embeddings):
    input_shape = batch.shape
    flat = jnp.reshape(batch, (ROWS, EMB_DIM))
    grid = ROWS // TILE
    enc, quant = pl.pallas_call(
        _vq_body,
        grid=(grid,),
        in_specs=[
            pl.BlockSpec((TILE, EMB_DIM), lambda i: (i, 0)),
            pl.BlockSpec((EMB_DIM, NUM_EMB), lambda i: (0, 0)),
        ],
        out_specs=[
            pl.BlockSpec((TILE, NUM_EMB), lambda i: (i, 0)),
            pl.BlockSpec((TILE, EMB_DIM), lambda i: (i, 0)),
        ],
        out_shape=[
            jax.ShapeDtypeStruct((ROWS, NUM_EMB), jnp.float32),
            jax.ShapeDtypeStruct((ROWS, EMB_DIM), jnp.float32),
        ],
    )(flat, embeddings)
    return (enc, jnp.reshape(quant, input_shape))


# speedup vs baseline: 1.7312x; 1.0512x over previous
import jax
import jax.numpy as jnp
from jax.experimental import pallas as pl

NUM_EMB = 1024
EMB_DIM = 256
ROWS = 16 * 1024
TILE = 1024


def _vq_body(x_ref, e_ref, enc_ref, out_ref):
    x = x_ref[:]
    emb = e_ref[:]
    sim = jnp.dot(x, emb, preferred_element_type=jnp.float32)
    e2 = jnp.sum(emb * emb, axis=0)
    x2 = jnp.sum(x * x, axis=1, keepdims=True)
    dist = x2 + e2[None, :] - 2.0 * sim
    idx = jnp.argmin(dist, axis=1)[:, None]
    iota = jax.lax.broadcasted_iota(jnp.int32, dist.shape, 1)
    onehot = (iota == idx).astype(jnp.float32)
    enc_ref[:] = onehot
    out_ref[:] = jax.lax.dot_general(
        onehot, emb, (((1,), (1,)), ((), ())),
        preferred_element_type=jnp.float32)


def kernel(batch, embeddings):
    input_shape = batch.shape
    flat = jnp.reshape(batch, (ROWS, EMB_DIM))
    grid = ROWS // TILE
    enc, quant = pl.pallas_call(
        _vq_body,
        grid=(grid,),
        in_specs=[
            pl.BlockSpec((TILE, EMB_DIM), lambda i: (i, 0)),
            pl.BlockSpec((EMB_DIM, NUM_EMB), lambda i: (0, 0)),
        ],
        out_specs=[
            pl.BlockSpec((TILE, NUM_EMB), lambda i: (i, 0)),
            pl.BlockSpec((TILE, EMB_DIM), lambda i: (i, 0)),
        ],
        out_shape=[
            jax.ShapeDtypeStruct((ROWS, NUM_EMB), jnp.float32),
            jax.ShapeDtypeStruct((ROWS, EMB_DIM), jnp.float32),
        ],
    )(flat, embeddings)
    return (enc, jnp.reshape(quant, input_shape))
